# baseline (speedup 1.0000x reference)
import jax
import jax.numpy as jnp
from jax import lax
from jax.experimental import pallas as pl
from jax.experimental.pallas import tpu as pltpu
from jax.experimental.pallas import tpu_sc as plsc

NUM_TIMES = 16
NUM_USERS = 2048
NUM_SERVICES = 1024
BATCH = 8192
K_TOP = 50
K_PAD = 64
SENTINEL = 1.0e38
VALID_CUT = 1.0e30

NC = 2
NS = 16
L = 16
NW = NC * NS
QPW = BATCH // NW
NB = QPW // L


_TOPK_BLK = 256


def _topk_body(sim_ref, tv_ref, ti_ref, work_ref):
    work_ref[...] = sim_ref[...]
    tv_ref[...] = jnp.zeros(tv_ref.shape, tv_ref.dtype)
    ti_ref[...] = jnp.zeros(ti_ref.shape, ti_ref.dtype)
    iota = lax.broadcasted_iota(jnp.int32, (_TOPK_BLK, NUM_USERS), 1)

    for it in range(K_TOP):
        w = work_ref[...]
        m = jnp.max(w, axis=1, keepdims=True)
        im = jnp.min(
            jnp.where(w == m, iota, jnp.int32(1 << 30)), axis=1, keepdims=True
        )
        tv_ref[:, it : it + 1] = m
        ti_ref[:, it : it + 1] = im
        work_ref[...] = jnp.where(iota == im, -jnp.inf, w)


def _topk_per_user(user_sim):
    grid = NUM_USERS // _TOPK_BLK
    tv, ti = pl.pallas_call(
        _topk_body,
        grid=(grid,),
        in_specs=[pl.BlockSpec((_TOPK_BLK, NUM_USERS), lambda i: (i, 0))],
        out_specs=[
            pl.BlockSpec((_TOPK_BLK, K_PAD), lambda i: (i, 0)),
            pl.BlockSpec((_TOPK_BLK, K_PAD), lambda i: (i, 0)),
        ],
        out_shape=[
            jax.ShapeDtypeStruct((NUM_USERS, K_PAD), jnp.float32),
            jax.ShapeDtypeStruct((NUM_USERS, K_PAD), jnp.int32),
        ],
        scratch_shapes=[pltpu.VMEM((_TOPK_BLK, NUM_USERS), jnp.float32)],
    )(user_sim)
    return tv, ti


def _sc_agg_body(
    t_hbm, u_hbm, s_hbm, tv_hbm, ti_hbm, p_hbm, avg_hbm,
    out_hbm,
    u_v, s_v, t_v, bidx_v, stv, base_v, nidx, fidx, tib, tvb, pq, outb,
    sem_b, sem_ti, sem_tv, sem_p,
):

    wid = lax.axis_index("s") * NC + lax.axis_index("c")
    q0 = wid * QPW

    pltpu.sync_copy(u_hbm.at[pl.ds(q0, QPW)], u_v)
    pltpu.sync_copy(s_hbm.at[pl.ds(q0, QPW)], s_v)
    pltpu.sync_copy(t_hbm.at[pl.ds(q0, QPW)], t_v)

    def prep_blk(b, _):
        s16 = s_v[pl.ds(b * L, L)]
        t16 = t_v[pl.ds(b * L, L)]
        u16 = u_v[pl.ds(b * L, L)]
        bidx_v[pl.ds(b * L, L)] = u16 * NUM_SERVICES + s16
        stv[pl.ds(b * L, L)] = t16 * NUM_SERVICES + s16
        return 0

    lax.fori_loop(0, NB, prep_blk, 0)

    c_b = pltpu.async_copy(avg_hbm.at[bidx_v], base_v, sem_b)

    def nidx_blk(b, _):
        u16 = u_v[pl.ds(b * L, L)] * K_PAD

        def nidx_k(k, _):
            nidx[pl.ds(k * QPW + b * L, L)] = u16 + k
            return 0

        lax.fori_loop(0, K_PAD, nidx_k, 0)
        return 0

    lax.fori_loop(0, NB, nidx_blk, 0)

    c_ti = pltpu.async_copy(ti_hbm.at[nidx], tib, sem_ti)
    c_tv = pltpu.async_copy(tv_hbm.at[nidx], tvb, sem_tv)
    c_ti.wait()

    def fidx_blk(b, _):
        st16 = stv[pl.ds(b * L, L)]

        def fidx_k(k, _):
            off = k * QPW + b * L
            fidx[pl.ds(off, L)] = (
                tib[pl.ds(off, L)] * (NUM_SERVICES * NUM_TIMES) + st16
            )
            return 0

        lax.fori_loop(0, K_PAD, fidx_k, 0)
        return 0

    lax.fori_loop(0, NB, fidx_blk, 0)

    c_p = pltpu.async_copy(p_hbm.at[fidx], pq, sem_p)
    c_tv.wait()
    c_p.wait()
    c_b.wait()

    def acc_blk(b, _):
        def acc_k(k, carry):
            dev, sim = carry
            off = k * QPW + b * L
            p = pq[pl.ds(off, L)]
            val = tvb[pl.ds(off, L)]
            valid = p < VALID_CUT
            w = jnp.where(valid, val, 0.0)
            pz = jnp.where(valid, p, 0.0)
            return (dev + w * pz, sim + w)

        dev, sim = lax.fori_loop(
            0, K_PAD, acc_k,
            (jnp.zeros((L,), jnp.float32), jnp.zeros((L,), jnp.float32)),
        )
        ok = sim > 0.0
        devf = jnp.where(ok, dev / jnp.where(ok, sim, 1.0), 0.0)
        outb[pl.ds(b * L, L)] = jnp.maximum(
            base_v[pl.ds(b * L, L)] + devf, 0.0
        )
        return 0

    lax.fori_loop(0, NB, acc_blk, 0)
    pltpu.sync_copy(outb, out_hbm.at[pl.ds(q0, QPW)])


def _sc_aggregate(t_ids, u_ids, s_ids, tv_flat, ti_flat, p_flat, avg_flat):
    mesh = plsc.VectorSubcoreMesh(
        core_axis_name="c", subcore_axis_name="s",
        num_cores=NC, num_subcores=NS,
    )
    run = pl.kernel(
        _sc_agg_body,
        out_type=jax.ShapeDtypeStruct((BATCH,), jnp.float32),
        mesh=mesh,
        scratch_types=[
            pltpu.VMEM((QPW,), jnp.int32),
            pltpu.VMEM((QPW,), jnp.int32),
            pltpu.VMEM((QPW,), jnp.int32),
            pltpu.VMEM((QPW,), jnp.int32),
            pltpu.VMEM((QPW,), jnp.int32),
            pltpu.VMEM((QPW,), jnp.float32),
            pltpu.VMEM((QPW * K_PAD,), jnp.int32),
            pltpu.VMEM((QPW * K_PAD,), jnp.int32),
            pltpu.VMEM((QPW * K_PAD,), jnp.int32),
            pltpu.VMEM((QPW * K_PAD,), jnp.float32),
            pltpu.VMEM((QPW * K_PAD,), jnp.float32),
            pltpu.VMEM((QPW,), jnp.float32),
            pltpu.SemaphoreType.DMA,
            pltpu.SemaphoreType.DMA,
            pltpu.SemaphoreType.DMA,
            pltpu.SemaphoreType.DMA,
        ],
    )
    return run(t_ids, u_ids, s_ids, tv_flat, ti_flat, p_flat, avg_flat)


@jax.jit
def kernel(x, qos_matrix, mask_matrix, avg_qos, user_sim):
    t_ids = x[:, 0]
    u_ids = x[:, 1]
    s_ids = x[:, 2]
    p3 = jnp.where(
        mask_matrix, qos_matrix - avg_qos[:, :, None], jnp.float32(SENTINEL)
    )
    p_flat = jnp.transpose(p3, (0, 2, 1)).reshape(-1)
    tv, ti = _topk_per_user(user_sim)
    return _sc_aggregate(
        t_ids, u_ids, s_ids, tv.reshape(-1), ti.reshape(-1), p_flat,
        avg_qos.reshape(-1)
    )


# speedup vs baseline: 1.0329x; 1.0329x over previous
import jax
import jax.numpy as jnp
from jax import lax
from jax.experimental import pallas as pl
from jax.experimental.pallas import tpu as pltpu
from jax.experimental.pallas import tpu_sc as plsc

NUM_TIMES = 16
NUM_USERS = 2048
NUM_SERVICES = 1024
BATCH = 8192
K_TOP = 50
K_PAD = 56
SENTINEL = 1.0e38
VALID_CUT = 1.0e30

NC = 2
NS = 16
L = 16
NW = NC * NS
QPW = BATCH // NW
NB = QPW // L


_TOPK_BLK = 256


def _topk_body(sim_ref, tv_ref, ti_ref, work_ref):
    work_ref[...] = sim_ref[...]
    tv_ref[...] = jnp.zeros(tv_ref.shape, tv_ref.dtype)
    ti_ref[...] = jnp.zeros(ti_ref.shape, ti_ref.dtype)
    iota = lax.broadcasted_iota(jnp.int32, (_TOPK_BLK, NUM_USERS), 1)

    for it in range(K_TOP):
        w = work_ref[...]
        m = jnp.max(w, axis=1, keepdims=True)
        im = jnp.min(
            jnp.where(w == m, iota, jnp.int32(1 << 30)), axis=1, keepdims=True
        )
        tv_ref[:, it : it + 1] = m
        ti_ref[:, it : it + 1] = im
        work_ref[...] = jnp.where(iota == im, -jnp.inf, w)


def _topk_per_user(user_sim):
    grid = NUM_USERS // _TOPK_BLK
    tv, ti = pl.pallas_call(
        _topk_body,
        grid=(grid,),
        in_specs=[pl.BlockSpec((_TOPK_BLK, NUM_USERS), lambda i: (i, 0))],
        out_specs=[
            pl.BlockSpec((_TOPK_BLK, K_PAD), lambda i: (i, 0)),
            pl.BlockSpec((_TOPK_BLK, K_PAD), lambda i: (i, 0)),
        ],
        out_shape=[
            jax.ShapeDtypeStruct((NUM_USERS, K_PAD), jnp.float32),
            jax.ShapeDtypeStruct((NUM_USERS, K_PAD), jnp.int32),
        ],
        scratch_shapes=[pltpu.VMEM((_TOPK_BLK, NUM_USERS), jnp.float32)],
    )(user_sim)
    return tv, ti


def _sc_agg_body(
    t_hbm, u_hbm, s_hbm, tv_hbm, ti_hbm, p_hbm, avg_hbm,
    out_hbm,
    u_v, s_v, t_v, bidx_v, stv, base_v, nidx, fidx, tib, tvb, pq, outb,
    sem_b, sem_ti, sem_tv, sem_p,
):

    wid = lax.axis_index("s") * NC + lax.axis_index("c")
    q0 = wid * QPW

    pltpu.sync_copy(u_hbm.at[pl.ds(q0, QPW)], u_v)
    pltpu.sync_copy(s_hbm.at[pl.ds(q0, QPW)], s_v)
    pltpu.sync_copy(t_hbm.at[pl.ds(q0, QPW)], t_v)

    def prep_blk(b, _):
        s16 = s_v[pl.ds(b * L, L)]
        t16 = t_v[pl.ds(b * L, L)]
        u16 = u_v[pl.ds(b * L, L)]
        bidx_v[pl.ds(b * L, L)] = u16 * NUM_SERVICES + s16
        stv[pl.ds(b * L, L)] = t16 * NUM_SERVICES + s16
        return 0

    lax.fori_loop(0, NB, prep_blk, 0)

    c_b = pltpu.async_copy(avg_hbm.at[bidx_v], base_v, sem_b)

    def nidx_blk(b, _):
        u16 = u_v[pl.ds(b * L, L)] * K_PAD

        def nidx_k(k, _):
            nidx[pl.ds(k * QPW + b * L, L)] = u16 + k
            return 0

        lax.fori_loop(0, K_PAD, nidx_k, 0)
        return 0

    lax.fori_loop(0, NB, nidx_blk, 0)

    c_ti = pltpu.async_copy(ti_hbm.at[nidx], tib, sem_ti)
    c_tv = pltpu.async_copy(tv_hbm.at[nidx], tvb, sem_tv)
    c_ti.wait()

    def fidx_blk(b, _):
        st16 = stv[pl.ds(b * L, L)]

        def fidx_k(k, _):
            off = k * QPW + b * L
            fidx[pl.ds(off, L)] = (
                tib[pl.ds(off, L)] * (NUM_SERVICES * NUM_TIMES) + st16
            )
            return 0

        lax.fori_loop(0, K_PAD, fidx_k, 0)
        return 0

    lax.fori_loop(0, NB, fidx_blk, 0)

    c_p = pltpu.async_copy(p_hbm.at[fidx], pq, sem_p)
    c_tv.wait()
    c_p.wait()
    c_b.wait()

    def acc_blk(b, _):
        def acc_k(k, carry):
            dev, sim = carry
            off = k * QPW + b * L
            p = pq[pl.ds(off, L)]
            val = tvb[pl.ds(off, L)]
            valid = p < VALID_CUT
            w = jnp.where(valid, val, 0.0)
            pz = jnp.where(valid, p, 0.0)
            return (dev + w * pz, sim + w)

        dev, sim = lax.fori_loop(
            0, K_PAD, acc_k,
            (jnp.zeros((L,), jnp.float32), jnp.zeros((L,), jnp.float32)),
        )
        ok = sim > 0.0
        devf = jnp.where(ok, dev / jnp.where(ok, sim, 1.0), 0.0)
        outb[pl.ds(b * L, L)] = jnp.maximum(
            base_v[pl.ds(b * L, L)] + devf, 0.0
        )
        return 0

    lax.fori_loop(0, NB, acc_blk, 0)
    pltpu.sync_copy(outb, out_hbm.at[pl.ds(q0, QPW)])


def _sc_aggregate(t_ids, u_ids, s_ids, tv_flat, ti_flat, p_flat, avg_flat):
    mesh = plsc.VectorSubcoreMesh(
        core_axis_name="c", subcore_axis_name="s",
        num_cores=NC, num_subcores=NS,
    )
    run = pl.kernel(
        _sc_agg_body,
        out_type=jax.ShapeDtypeStruct((BATCH,), jnp.float32),
        mesh=mesh,
        scratch_types=[
            pltpu.VMEM((QPW,), jnp.int32),
            pltpu.VMEM((QPW,), jnp.int32),
            pltpu.VMEM((QPW,), jnp.int32),
            pltpu.VMEM((QPW,), jnp.int32),
            pltpu.VMEM((QPW,), jnp.int32),
            pltpu.VMEM((QPW,), jnp.float32),
            pltpu.VMEM((QPW * K_PAD,), jnp.int32),
            pltpu.VMEM((QPW * K_PAD,), jnp.int32),
            pltpu.VMEM((QPW * K_PAD,), jnp.int32),
            pltpu.VMEM((QPW * K_PAD,), jnp.float32),
            pltpu.VMEM((QPW * K_PAD,), jnp.float32),
            pltpu.VMEM((QPW,), jnp.float32),
            pltpu.SemaphoreType.DMA,
            pltpu.SemaphoreType.DMA,
            pltpu.SemaphoreType.DMA,
            pltpu.SemaphoreType.DMA,
        ],
    )
    return run(t_ids, u_ids, s_ids, tv_flat, ti_flat, p_flat, avg_flat)


@jax.jit
def kernel(x, qos_matrix, mask_matrix, avg_qos, user_sim):
    t_ids = x[:, 0]
    u_ids = x[:, 1]
    s_ids = x[:, 2]
    p3 = jnp.where(
        mask_matrix, qos_matrix - avg_qos[:, :, None], jnp.float32(SENTINEL)
    )
    p_flat = jnp.transpose(p3, (0, 2, 1)).reshape(-1)
    tv, ti = _topk_per_user(user_sim)
    return _sc_aggregate(
        t_ids, u_ids, s_ids, tv.reshape(-1), ti.reshape(-1), p_flat,
        avg_qos.reshape(-1)
    )
